# baseline (speedup 1.0000x reference)
import functools

import jax
import jax.numpy as jnp
from jax import lax
from jax.experimental import pallas as pl
from jax.experimental.pallas import tpu as pltpu
from jax.experimental.pallas import tpu_sc as plsc

NUM_EMBEDDINGS = 1000000
EMBEDDING_DIM = 32
BATCH = 16384
FIELDS = 26

_TOTAL = BATCH * FIELDS
_NW = 32
_PER_W = _TOTAL // _NW
_CHUNK = 1664
_NCHUNK = _PER_W // _CHUNK
_NBUF = 2

_FULL_COLS = NUM_EMBEDDINGS // 128
_REM = NUM_EMBEDDINGS - _FULL_COLS * 128
_COLS_PER_W = _FULL_COLS // _NW
_COLS_EXTRA = _FULL_COLS - _COLS_PER_W * _NW

assert _PER_W % _CHUNK == 0 and _CHUNK % 8 == 0


def _worker_id():
    return lax.axis_index("s") * 2 + lax.axis_index("c")


def _make_detranspose():
    mesh = plsc.VectorSubcoreMesh(core_axis_name="c", subcore_axis_name="s")

    @functools.partial(
        pl.kernel,
        mesh=mesh,
        out_type=jax.ShapeDtypeStruct((NUM_EMBEDDINGS * EMBEDDING_DIM // 128,
                                       128), jnp.float32),
        scratch_types=[
            pltpu.VMEM((2, 32, 128), jnp.float32),
            pltpu.VMEM((2, 32, 128), jnp.float32),
            pltpu.VMEM((16, 128), jnp.float32),
            [pltpu.SemaphoreType.DMA] * 2,
            [pltpu.SemaphoreType.DMA] * 2,
        ],
        compiler_params=pltpu.CompilerParams(use_tc_tiling_on_sc=True,
                                             needs_layout_passes=False,
                                             disable_bounds_checks=True),
    )
    def detr_kernel(wt_hbm, wtail_hbm, flat_hbm, inbuf, stage, rembuf,
                    isems, osems):
        wid = _worker_id()
        n_cols = jnp.where(wid < _COLS_EXTRA, _COLS_PER_W + 1, _COLS_PER_W)
        col0 = wid * _COLS_PER_W + jnp.minimum(wid, _COLS_EXTRA)
        end = col0 + n_cols
        lane = lax.iota(jnp.int32, 16)
        jmod = [(lane + d) % 16 for d in range(16)]
        jconst = [[jmod[d] + jb * 16 for jb in range(2)] for d in range(16)]
        ivec = [lane + l * 16 for l in range(8)]
        dbase = [[lane * 32 + jmod[d] + jb * 16 for jb in range(2)]
                 for d in range(16)]
        scol = [[lax.bitwise_and(dbase[d][jb], 127) for jb in range(2)]
                for d in range(16)]
        srow0 = [[lax.shift_right_logical(dbase[d][jb], 7) for jb in range(2)]
                 for d in range(16)]

        def start_in(col, b):
            return pltpu.async_copy(
                wt_hbm.at[:, pl.ds(col * 128, 128)], inbuf.at[b], isems[b])

        def wait_in(col, b):
            pltpu.make_async_copy(
                wt_hbm.at[:, pl.ds(col * 128, 128)], inbuf.at[b],
                isems[b]).wait()

        def start_out(col, b):
            return pltpu.async_copy(
                stage.at[b], flat_hbm.at[pl.ds(col * 32, 32), :], osems[b])

        def wait_out(col, b):
            pltpu.make_async_copy(
                stage.at[b], flat_hbm.at[pl.ds(col * 32, 32), :],
                osems[b]).wait()

        start_in(col0, 0)

        def body(kk, _):
            for i in (0, 1):
                col = col0 + kk * 2 + i
                b = i

                @pl.when(col < end)
                def _():
                    @pl.when(col + 1 < end)
                    def _():
                        start_in(col + 1, 1 - b)

                    wait_in(col, b)

                    @pl.when(col - 2 >= col0)
                    def _():
                        wait_out(col - 2, b)

                    def lbody(l, _):
                        iv = lane + l * 16
                        for jb in range(2):
                            for d in range(16):
                                v = plsc.load_gather(
                                    inbuf.at[b],
                                    [jconst[d][jb], iv])
                                plsc.store_scatter(
                                    stage.at[b],
                                    [srow0[d][jb] + l * 4, scol[d][jb]], v)
                        return 0

                    lax.fori_loop(0, 8, lbody, 0)
                    start_out(col, b)
            return 0

        lax.fori_loop(0, (_COLS_PER_W + 2) // 2, body, 0)

        @pl.when(n_cols % 2 == 0)
        def _():
            wait_out(end - 2, 0)
            wait_out(end - 1, 1)

        @pl.when(n_cols % 2 == 1)
        def _():
            wait_out(end - 2, 1)
            wait_out(end - 1, 0)

        @pl.when(wid == _NW - 1)
        def _():
            pltpu.sync_copy(wtail_hbm, rembuf)
            pltpu.sync_copy(rembuf,
                            flat_hbm.at[pl.ds(_FULL_COLS * 32, 16), :])

    return detr_kernel


def _make_gather_flat():
    mesh = plsc.VectorSubcoreMesh(core_axis_name="c", subcore_axis_name="s")

    @functools.partial(
        pl.kernel,
        mesh=mesh,
        out_type=jax.ShapeDtypeStruct((_TOTAL, EMBEDDING_DIM), jnp.float32),
        scratch_types=[
            pltpu.VMEM((_PER_W,), jnp.int32),
            pltpu.VMEM((2, _CHUNK, EMBEDDING_DIM), jnp.float32),
            [pltpu.SemaphoreType.DMA] * 2,
            [pltpu.SemaphoreType.DMA] * 2,
        ],
        compiler_params=pltpu.CompilerParams(use_tc_tiling_on_sc=False,
                                             needs_layout_passes=False),
    )
    def gatf_kernel(table_hbm, idx_hbm, out_hbm, idx_v, rows_v, gsems, osems):
        wid = _worker_id()
        base = wid * _PER_W
        pltpu.sync_copy(idx_hbm.at[pl.ds(base, _PER_W)], idx_v)

        def start_gather(k, b):
            pltpu.async_copy(
                table_hbm.at[idx_v.at[pl.ds(k * _CHUNK, _CHUNK)]],
                rows_v.at[b], gsems[b])

        def wait_gather(k, b):
            pltpu.make_async_copy(
                table_hbm.at[idx_v.at[pl.ds(k * _CHUNK, _CHUNK)]],
                rows_v.at[b], gsems[b]).wait()

        def start_out(k, b):
            pltpu.async_copy(
                rows_v.at[b],
                out_hbm.at[pl.ds(base + k * _CHUNK, _CHUNK)], osems[b])

        def wait_out(k, b):
            pltpu.make_async_copy(
                rows_v.at[b],
                out_hbm.at[pl.ds(base + k * _CHUNK, _CHUNK)], osems[b]).wait()

        start_gather(0, 0)
        for k in range(_NCHUNK):
            b = k % 2
            if k + 1 < _NCHUNK:
                if k - 1 >= 0:
                    wait_out(k - 1, 1 - b)
                start_gather(k + 1, 1 - b)
            wait_gather(k, b)
            start_out(k, b)
        wait_out(_NCHUNK - 2, _NCHUNK % 2)
        wait_out(_NCHUNK - 1, 1 - _NCHUNK % 2)

    return gatf_kernel


_SLABS_PER_W = (BATCH // 128) // _NW
_UNITS_PER_W = _SLABS_PER_W * FIELDS
_TILE_ROWS = _TOTAL * EMBEDDING_DIM // 128


def _make_gather_t():
    mesh = plsc.VectorSubcoreMesh(core_axis_name="c", subcore_axis_name="s")

    @functools.partial(
        pl.kernel,
        mesh=mesh,
        out_type=jax.ShapeDtypeStruct((_TILE_ROWS, 128), jnp.float32),
        scratch_types=[
            pltpu.VMEM((_PER_W,), jnp.int32),
            pltpu.VMEM((4, 128), jnp.int32),
            pltpu.VMEM((4, 128, EMBEDDING_DIM), jnp.float32),
            pltpu.VMEM((4, 32, 128), jnp.float32),
            [pltpu.SemaphoreType.DMA] * 4,
            [pltpu.SemaphoreType.DMA] * 4,
        ],
        compiler_params=pltpu.CompilerParams(use_tc_tiling_on_sc=False,
                                             needs_layout_passes=False),
    )
    def gat_kernel(table_hbm, idx_hbm, out_hbm, idx_v, uidx, rows_v, stage,
                   gsems, osems):
        wid = _worker_id()
        base = wid * _PER_W
        slab0 = wid * _SLABS_PER_W
        pltpu.sync_copy(idx_hbm.at[pl.ds(base, _PER_W)], idx_v)
        lane = lax.iota(jnp.int32, 16)
        jmod = [(lane + d) % 16 for d in range(16)]
        rvec = [lane + l * 16 for l in range(8)]

        def extract_idx(u, b):
            slab = u // FIELDS
            f = u % FIELDS
            for g in range(8):
                src = (slab * 128 + g * 16 + lane) * FIELDS + f
                v = plsc.load_gather(idx_v, [src])
                uidx[b, pl.ds(g * 16, 16)] = v

        def start_gather(b):
            return pltpu.async_copy(
                table_hbm.at[uidx.at[b]], rows_v.at[b], gsems[b])

        def wait_gather(b):
            pltpu.make_async_copy(
                table_hbm.at[uidx.at[b]], rows_v.at[b], gsems[b]).wait()

        def transpose(b):
            def lbody(l, _):
                rv = lane + l * 16
                for jb in range(2):
                    for d in range(16):
                        v = plsc.load_gather(
                            rows_v.at[b], [rv, jmod[d] + jb * 16])
                        plsc.store_scatter(
                            stage.at[b], [jmod[d] + jb * 16, rv], v)
                return 0

            lax.fori_loop(0, 8, lbody, 0)

        def out_row0(u, tj):
            slab = u // FIELDS
            f = u % FIELDS
            return ((f * 4 + tj) * 128 + slab0 + slab) * 8

        def start_out(u, b):
            for tj in range(4):
                pltpu.async_copy(
                    stage.at[b, pl.ds(tj * 8, 8)],
                    out_hbm.at[pl.ds(out_row0(u, tj), 8), :], osems[b])

        def wait_out(u, b):
            for tj in range(4):
                pltpu.make_async_copy(
                    stage.at[b, pl.ds(tj * 8, 8)],
                    out_hbm.at[pl.ds(out_row0(u, tj), 8), :], osems[b]).wait()

        for u0 in (0, 1):
            extract_idx(u0, u0)
            start_gather(u0)

        def body(kk, _):
            for i in range(4):
                u = kk * 4 + i
                b = i
                wait_gather(b)

                b2 = (i + 2) % 4

                @pl.when(u + 2 < _UNITS_PER_W)
                def _():
                    extract_idx(u + 2, b2)
                    start_gather(b2)

                @pl.when(u - 4 >= 0)
                def _():
                    wait_out(u - 4, b)

                transpose(b)
                start_out(u, b)
            return 0

        lax.fori_loop(0, _UNITS_PER_W // 4, body, 0)
        for u in range(_UNITS_PER_W - 4, _UNITS_PER_W):
            wait_out(u, u % 4)

    return gat_kernel


def _make_format():
    mesh = plsc.VectorSubcoreMesh(core_axis_name="c", subcore_axis_name="s")
    n_units = FIELDS * 4 * 4
    per_w = n_units // _NW

    @functools.partial(
        pl.kernel,
        mesh=mesh,
        out_type=jax.ShapeDtypeStruct((FIELDS, EMBEDDING_DIM, BATCH),
                                      jnp.float32),
        scratch_types=[
            pltpu.VMEM((2, 8, 4096), jnp.float32),
            [pltpu.SemaphoreType.DMA] * 2,
            [pltpu.SemaphoreType.DMA] * 2,
        ],
        compiler_params=pltpu.CompilerParams(use_tc_tiling_on_sc=True,
                                             needs_layout_passes=False),
    )
    def fmt_kernel(src_hbm, out_hbm, vbuf, isems, osems):
        wid = _worker_id()
        u0 = wid * per_w

        def decode(u):
            f = u // 16
            r = u % 16
            return f, r // 4, r % 4

        def start_in(u, b):
            return pltpu.async_copy(src_hbm.at[u], vbuf.at[b], isems[b])

        def wait_in(u, b):
            pltpu.make_async_copy(src_hbm.at[u], vbuf.at[b], isems[b]).wait()

        def start_out(u, b):
            f, tj, q = decode(u)
            return pltpu.async_copy(
                vbuf.at[b],
                out_hbm.at[f, pl.ds(tj * 8, 8), pl.ds(q * 4096, 4096)],
                osems[b])

        def wait_out(u, b):
            f, tj, q = decode(u)
            pltpu.make_async_copy(
                vbuf.at[b],
                out_hbm.at[f, pl.ds(tj * 8, 8), pl.ds(q * 4096, 4096)],
                osems[b]).wait()

        start_in(u0, 0)
        for k in range(per_w):
            u = u0 + k
            b = k % 2
            if k + 1 < per_w:
                if k - 1 >= 0:
                    wait_out(u - 1, 1 - b)
                start_in(u + 1, 1 - b)
            wait_in(u, b)
            start_out(u, b)
        wait_out(u0 + per_w - 2, (per_w - 2) % 2)
        wait_out(u0 + per_w - 1, (per_w - 1) % 2)

    return fmt_kernel


_DETR = _make_detranspose()
_GAT = _make_gather_t()


@jax.jit
def kernel(x, weight):
    idx = x.reshape(-1).astype(jnp.int32)
    wtail = weight[_FULL_COLS * 128:].reshape(16, 128)
    flat_table = _DETR(weight.T, wtail)
    table = flat_table.reshape(NUM_EMBEDDINGS, EMBEDDING_DIM)
    interm = _GAT(table, idx)
    v5 = interm.reshape(FIELDS, 4, BATCH // 128, 8, 128)
    out_feb = v5.transpose(0, 1, 3, 2, 4).reshape(FIELDS, EMBEDDING_DIM,
                                                  BATCH)
    return jnp.transpose(out_feb, (2, 0, 1))


# speedup vs baseline: 1.0222x; 1.0222x over previous
import functools

import jax
import jax.numpy as jnp
from jax import lax
from jax.experimental import pallas as pl
from jax.experimental.pallas import tpu as pltpu
from jax.experimental.pallas import tpu_sc as plsc

NUM_EMBEDDINGS = 1000000
EMBEDDING_DIM = 32
BATCH = 16384
FIELDS = 26

_TOTAL = BATCH * FIELDS
_NW = 32
_PER_W = _TOTAL // _NW

_FULL_COLS = NUM_EMBEDDINGS // 128
_COLS_PER_W = _FULL_COLS // _NW
_COLS_EXTRA = _FULL_COLS - _COLS_PER_W * _NW


def _worker_id():
    return lax.axis_index("s") * 2 + lax.axis_index("c")


def _make_detranspose():
    mesh = plsc.VectorSubcoreMesh(core_axis_name="c", subcore_axis_name="s")

    @functools.partial(
        pl.kernel,
        mesh=mesh,
        out_type=jax.ShapeDtypeStruct((NUM_EMBEDDINGS * EMBEDDING_DIM // 128,
                                       128), jnp.float32),
        scratch_types=[
            pltpu.VMEM((2, 32, 128), jnp.float32),
            pltpu.VMEM((2, 32, 128), jnp.float32),
            pltpu.VMEM((16, 128), jnp.float32),
            [pltpu.SemaphoreType.DMA] * 2,
            [pltpu.SemaphoreType.DMA] * 2,
        ],
        compiler_params=pltpu.CompilerParams(use_tc_tiling_on_sc=True,
                                             needs_layout_passes=False,
                                             disable_bounds_checks=True),
    )
    def detr_kernel(wt_hbm, wtail_hbm, flat_hbm, inbuf, stage, rembuf,
                    isems, osems):
        wid = _worker_id()
        n_cols = jnp.where(wid < _COLS_EXTRA, _COLS_PER_W + 1, _COLS_PER_W)
        col0 = wid * _COLS_PER_W + jnp.minimum(wid, _COLS_EXTRA)
        end = col0 + n_cols
        lane = lax.iota(jnp.int32, 16)
        jmod = [(lane + d) % 16 for d in range(16)]
        jconst = [[jmod[d] + jb * 16 for jb in range(2)] for d in range(16)]
        dbase = [[lane * 32 + jmod[d] + jb * 16 for jb in range(2)]
                 for d in range(16)]
        scol = [[lax.bitwise_and(dbase[d][jb], 127) for jb in range(2)]
                for d in range(16)]
        srow0 = [[lax.shift_right_logical(dbase[d][jb], 7) for jb in range(2)]
                 for d in range(16)]

        def start_in(col, b):
            return pltpu.async_copy(
                wt_hbm.at[:, pl.ds(col * 128, 128)], inbuf.at[b], isems[b])

        def wait_in(col, b):
            pltpu.make_async_copy(
                wt_hbm.at[:, pl.ds(col * 128, 128)], inbuf.at[b],
                isems[b]).wait()

        def start_out(col, b):
            return pltpu.async_copy(
                stage.at[b], flat_hbm.at[pl.ds(col * 32, 32), :], osems[b])

        def wait_out(col, b):
            pltpu.make_async_copy(
                stage.at[b], flat_hbm.at[pl.ds(col * 32, 32), :],
                osems[b]).wait()

        start_in(col0, 0)

        def body(kk, _):
            for i in (0, 1):
                col = col0 + kk * 2 + i
                b = i

                @pl.when(col < end)
                def _():
                    @pl.when(col + 1 < end)
                    def _():
                        start_in(col + 1, 1 - b)

                    wait_in(col, b)

                    @pl.when(col - 2 >= col0)
                    def _():
                        wait_out(col - 2, b)

                    def lbody(l, _):
                        iv = lane + l * 16
                        for jb in range(2):
                            for d in range(16):
                                v = plsc.load_gather(
                                    inbuf.at[b],
                                    [jconst[d][jb], iv])
                                plsc.store_scatter(
                                    stage.at[b],
                                    [srow0[d][jb] + l * 4, scol[d][jb]], v)
                        return 0

                    lax.fori_loop(0, 8, lbody, 0)
                    start_out(col, b)
            return 0

        lax.fori_loop(0, (_COLS_PER_W + 2) // 2, body, 0)

        @pl.when(n_cols % 2 == 0)
        def _():
            wait_out(end - 2, 0)
            wait_out(end - 1, 1)

        @pl.when(n_cols % 2 == 1)
        def _():
            wait_out(end - 2, 1)
            wait_out(end - 1, 0)

        @pl.when(wid == _NW - 1)
        def _():
            pltpu.sync_copy(wtail_hbm, rembuf)
            pltpu.sync_copy(rembuf,
                            flat_hbm.at[pl.ds(_FULL_COLS * 32, 16), :])

    return detr_kernel


_SLABS_PER_W = (BATCH // 128) // _NW
_UNITS_PER_W = _SLABS_PER_W * FIELDS
_TILE_ROWS = _TOTAL * EMBEDDING_DIM // 128


def _make_gather_t():
    mesh = plsc.VectorSubcoreMesh(core_axis_name="c", subcore_axis_name="s")

    @functools.partial(
        pl.kernel,
        mesh=mesh,
        out_type=jax.ShapeDtypeStruct((_TILE_ROWS, 128), jnp.float32),
        scratch_types=[
            pltpu.VMEM((_PER_W,), jnp.int32),
            pltpu.VMEM((2, 128), jnp.int32),
            pltpu.VMEM((2, 128, EMBEDDING_DIM), jnp.float32),
            pltpu.VMEM((2, 32, 128), jnp.float32),
            [pltpu.SemaphoreType.DMA] * 2,
            [pltpu.SemaphoreType.DMA] * 2,
        ],
        compiler_params=pltpu.CompilerParams(use_tc_tiling_on_sc=False,
                                             needs_layout_passes=False),
    )
    def gat_kernel(table_hbm, idx_hbm, out_hbm, idx_v, uidx, rows_v, stage,
                   gsems, osems):
        wid = _worker_id()
        base = wid * _PER_W
        slab0 = wid * _SLABS_PER_W
        pltpu.sync_copy(idx_hbm.at[pl.ds(base, _PER_W)], idx_v)
        lane = lax.iota(jnp.int32, 16)
        jmod = [(lane + d) % 16 for d in range(16)]

        def extract_idx(u, b):
            slab = u // FIELDS
            f = u % FIELDS
            for g in range(8):
                src = (slab * 128 + g * 16 + lane) * FIELDS + f
                v = plsc.load_gather(idx_v, [src])
                uidx[b, pl.ds(g * 16, 16)] = v

        def start_gather(b):
            return pltpu.async_copy(
                table_hbm.at[uidx.at[b]], rows_v.at[b], gsems[b])

        def wait_gather(b):
            pltpu.make_async_copy(
                table_hbm.at[uidx.at[b]], rows_v.at[b], gsems[b]).wait()

        def transpose(b):
            def lbody(l, _):
                rv = lane + l * 16
                for jb in range(2):
                    for d in range(16):
                        v = plsc.load_gather(
                            rows_v.at[b], [rv, jmod[d] + jb * 16])
                        plsc.store_scatter(
                            stage.at[b], [jmod[d] + jb * 16, rv], v)
                return 0

            lax.fori_loop(0, 8, lbody, 0)

        def out_row0(u, tj):
            slab = u // FIELDS
            f = u % FIELDS
            return ((f * 4 + tj) * 128 + slab0 + slab) * 8

        def start_out(u, b):
            for tj in range(4):
                pltpu.async_copy(
                    stage.at[b, pl.ds(tj * 8, 8)],
                    out_hbm.at[pl.ds(out_row0(u, tj), 8), :], osems[b])

        def wait_out(u, b):
            for tj in range(4):
                pltpu.make_async_copy(
                    stage.at[b, pl.ds(tj * 8, 8)],
                    out_hbm.at[pl.ds(out_row0(u, tj), 8), :], osems[b]).wait()

        extract_idx(0, 0)
        start_gather(0)

        def body(kk, _):
            for i in (0, 1):
                u = kk * 2 + i
                b = i

                @pl.when(u < _UNITS_PER_W)
                def _():
                    @pl.when(u + 1 < _UNITS_PER_W)
                    def _():
                        extract_idx(u + 1, 1 - b)
                        start_gather(1 - b)

                    wait_gather(b)

                    @pl.when(u - 2 >= 0)
                    def _():
                        wait_out(u - 2, b)

                    transpose(b)
                    start_out(u, b)
            return 0

        lax.fori_loop(0, _UNITS_PER_W // 2, body, 0)
        wait_out(_UNITS_PER_W - 2, 0)
        wait_out(_UNITS_PER_W - 1, 1)

    return gat_kernel


_DETR = _make_detranspose()
_GAT = _make_gather_t()


@jax.jit
def kernel(x, weight):
    idx = x.reshape(-1).astype(jnp.int32)
    wtail = weight[_FULL_COLS * 128:].reshape(16, 128)
    flat_table = _DETR(weight.T, wtail)
    table = flat_table.reshape(NUM_EMBEDDINGS, EMBEDDING_DIM)
    interm = _GAT(table, idx)
    v5 = interm.reshape(FIELDS, 4, BATCH // 128, 8, 128)
    out_feb = v5.transpose(0, 1, 3, 2, 4).reshape(FIELDS, EMBEDDING_DIM,
                                                  BATCH)
    return jnp.transpose(out_feb, (2, 0, 1))
